# baseline (speedup 1.0000x reference)
import functools

import jax
import jax.numpy as jnp
from jax import lax
from jax.experimental import pallas as pl
from jax.experimental.pallas import tpu as pltpu
from jax.experimental.pallas import tpu_sc as plsc

_N = 1048576
_N_SPECIES = 119
_TAB = 128
_NC, _NS, _L = 2, 16, 16
_NW = _NC * _NS
_PER_W = _N // _NW
_CHUNK = 8192
_NCH = _PER_W // _CHUNK


def _build():
    mesh = plsc.VectorSubcoreMesh(core_axis_name="c", subcore_axis_name="s")

    @functools.partial(
        pl.kernel,
        mesh=mesh,
        compiler_params=pltpu.CompilerParams(needs_layout_passes=False),
        out_type=jax.ShapeDtypeStruct((_N,), jnp.float32),
        scratch_types=[
            pltpu.VMEM((_N_SPECIES, 1), jnp.float32),
            pltpu.VMEM((_TAB,), jnp.float32),
            pltpu.VMEM((_CHUNK,), jnp.int32),
            pltpu.VMEM((_CHUNK,), jnp.int32),
            pltpu.VMEM((_CHUNK,), jnp.float32),
            pltpu.VMEM((_CHUNK,), jnp.float32),
            pltpu.VMEM((_CHUNK,), jnp.float32),
            pltpu.VMEM((_CHUNK,), jnp.float32),
            pltpu.SemaphoreType.DMA,
            pltpu.SemaphoreType.DMA,
            pltpu.SemaphoreType.DMA,
            pltpu.SemaphoreType.DMA,
        ],
    )
    def k(x_hbm, idx_hbm, shift_hbm, out_hbm, table2d_v, table_v,
          idx_v0, idx_v1, x_v0, x_v1, y_v0, y_v1,
          sem_in0, sem_in1, sem_out0, sem_out1):
        wid = lax.axis_index("s") * _NC + lax.axis_index("c")
        base = wid * _PER_W
        idx_bufs = (idx_v0, idx_v1)
        x_bufs = (x_v0, x_v1)
        y_bufs = (y_v0, y_v1)
        sem_in = (sem_in0, sem_in1)
        sem_out = (sem_out0, sem_out1)

        def start_in(c):
            sl = pl.ds(base + c * _CHUNK, _CHUNK)
            s = sem_in[c % 2]
            return (pltpu.async_copy(idx_hbm.at[sl], idx_bufs[c % 2], s),
                    pltpu.async_copy(x_hbm.at[sl], x_bufs[c % 2], s))

        in_flight = [start_in(0)]
        pltpu.sync_copy(shift_hbm, table2d_v)
        lane = lax.iota(jnp.int32, _L)
        zero16 = jnp.zeros((_L,), jnp.int32)
        for g in range(_TAB // _L):
            src = jnp.minimum(lane + g * _L, _N_SPECIES - 1)
            table_v[pl.ds(g * _L, _L)] = plsc.load_gather(
                table2d_v, [src, zero16])
        out_flight = [None, None]

        for c in range(_NCH):
            if c + 1 < _NCH:
                in_flight.append(start_in(c + 1))
            for d in in_flight.pop(0):
                d.wait()
            if out_flight[c % 2] is not None:
                out_flight[c % 2].wait()

            ib = idx_bufs[c % 2]
            xb = x_bufs[c % 2]
            yb = y_bufs[c % 2]

            @plsc.parallel_loop(0, _CHUNK, step=_L, unroll=8)
            def _(i):
                sl = pl.ds(i, _L)
                sv = plsc.load_gather(table_v, [ib[sl]])
                yb[sl] = xb[sl] - sv

            out_flight[c % 2] = pltpu.async_copy(
                yb, out_hbm.at[pl.ds(base + c * _CHUNK, _CHUNK)],
                sem_out[c % 2])

        for d in out_flight:
            if d is not None:
                d.wait()

    return k


_sc_kernel = _build()


def kernel(x, atomic_numbers, shift):
    return _sc_kernel(x, atomic_numbers.astype(jnp.int32), shift)


# speedup vs baseline: 1.0512x; 1.0512x over previous
import functools

import jax
import jax.numpy as jnp
from jax import lax
from jax.experimental import pallas as pl
from jax.experimental.pallas import tpu as pltpu
from jax.experimental.pallas import tpu_sc as plsc

_N = 1048576
_N_SPECIES = 119
_TAB = 128
_NC, _NS, _L = 2, 16, 16
_NW = _NC * _NS
_PER_W = _N // _NW
_CHUNK = 8192
_SCHED = (4096, 8192, 8192, 8192, 4096)
_OFFS = tuple(sum(_SCHED[:i]) for i in range(len(_SCHED)))
assert sum(_SCHED) == _PER_W
_NCH = len(_SCHED)


def _build():
    mesh = plsc.VectorSubcoreMesh(core_axis_name="c", subcore_axis_name="s")

    @functools.partial(
        pl.kernel,
        mesh=mesh,
        compiler_params=pltpu.CompilerParams(needs_layout_passes=False),
        out_type=jax.ShapeDtypeStruct((_N,), jnp.float32),
        scratch_types=[
            pltpu.VMEM((_TAB,), jnp.float32),
            pltpu.VMEM((_CHUNK,), jnp.int32),
            pltpu.VMEM((_CHUNK,), jnp.int32),
            pltpu.VMEM((_CHUNK,), jnp.float32),
            pltpu.VMEM((_CHUNK,), jnp.float32),
            pltpu.VMEM((_CHUNK,), jnp.float32),
            pltpu.VMEM((_CHUNK,), jnp.float32),
            pltpu.SemaphoreType.DMA,
            pltpu.SemaphoreType.DMA,
            pltpu.SemaphoreType.DMA,
            pltpu.SemaphoreType.DMA,
        ],
    )
    def k(x_hbm, idx_hbm, shift_hbm, out_hbm, table_v,
          idx_v0, idx_v1, x_v0, x_v1, y_v0, y_v1,
          sem_in0, sem_in1, sem_out0, sem_out1):
        wid = lax.axis_index("s") * _NC + lax.axis_index("c")
        base = wid * _PER_W
        idx_bufs = (idx_v0, idx_v1)
        x_bufs = (x_v0, x_v1)
        y_bufs = (y_v0, y_v1)
        sem_in = (sem_in0, sem_in1)
        sem_out = (sem_out0, sem_out1)

        def start_in(c):
            n = _SCHED[c]
            sl = pl.ds(base + _OFFS[c], n)
            s = sem_in[c % 2]
            return (pltpu.async_copy(idx_hbm.at[sl],
                                     idx_bufs[c % 2].at[pl.ds(0, n)], s),
                    pltpu.async_copy(x_hbm.at[sl],
                                     x_bufs[c % 2].at[pl.ds(0, n)], s))

        in_flight = [start_in(0)]
        pltpu.sync_copy(shift_hbm, table_v)
        out_flight = [None, None]

        for c in range(_NCH):
            if c + 1 < _NCH:
                in_flight.append(start_in(c + 1))
            for d in in_flight.pop(0):
                d.wait()
            if out_flight[c % 2] is not None:
                out_flight[c % 2].wait()

            ib = idx_bufs[c % 2]
            xb = x_bufs[c % 2]
            yb = y_bufs[c % 2]

            @plsc.parallel_loop(0, _SCHED[c], step=_L, unroll=8)
            def _(i):
                sl = pl.ds(i, _L)
                sv = plsc.load_gather(table_v, [ib[sl]])
                yb[sl] = xb[sl] - sv

            out_flight[c % 2] = pltpu.async_copy(
                yb.at[pl.ds(0, _SCHED[c])],
                out_hbm.at[pl.ds(base + _OFFS[c], _SCHED[c])],
                sem_out[c % 2])

        for d in out_flight:
            if d is not None:
                d.wait()

    return k


_sc_kernel = _build()


def kernel(x, atomic_numbers, shift):
    idx = atomic_numbers.astype(jnp.int32)
    table = jnp.pad(shift.reshape(-1), (0, _TAB - _N_SPECIES))
    return _sc_kernel(x, idx, table)


# speedup vs baseline: 1.0730x; 1.0208x over previous
import functools

import jax
import jax.numpy as jnp
from jax import lax
from jax.experimental import pallas as pl
from jax.experimental.pallas import tpu as pltpu
from jax.experimental.pallas import tpu_sc as plsc

_N = 1048576
_N_SPECIES = 119
_TAB = 128
_NC, _NS, _L = 2, 16, 16
_NW = _NC * _NS
_PER_W = _N // _NW
_CHUNK = 8192
_SCHED = (8192, 8192, 8192, 8192)
_OFFS = tuple(sum(_SCHED[:i]) for i in range(len(_SCHED)))
assert sum(_SCHED) == _PER_W
_NCH = len(_SCHED)


def _build():
    mesh = plsc.VectorSubcoreMesh(core_axis_name="c", subcore_axis_name="s")

    @functools.partial(
        pl.kernel,
        mesh=mesh,
        compiler_params=pltpu.CompilerParams(needs_layout_passes=False),
        out_type=jax.ShapeDtypeStruct((_N,), jnp.float32),
        scratch_types=[
            pltpu.VMEM((_TAB,), jnp.float32),
            pltpu.VMEM((_CHUNK,), jnp.int32),
            pltpu.VMEM((_CHUNK,), jnp.int32),
            pltpu.VMEM((_CHUNK,), jnp.float32),
            pltpu.VMEM((_CHUNK,), jnp.float32),
            pltpu.VMEM((_CHUNK,), jnp.float32),
            pltpu.VMEM((_CHUNK,), jnp.float32),
            pltpu.SemaphoreType.DMA,
            pltpu.SemaphoreType.DMA,
            pltpu.SemaphoreType.DMA,
            pltpu.SemaphoreType.DMA,
        ],
    )
    def k(x_hbm, idx_hbm, shift_hbm, out_hbm, table_v,
          idx_v0, idx_v1, x_v0, x_v1, y_v0, y_v1,
          sem_in0, sem_in1, sem_out0, sem_out1):
        wid = lax.axis_index("s") * _NC + lax.axis_index("c")
        base = wid * _PER_W
        idx_bufs = (idx_v0, idx_v1)
        x_bufs = (x_v0, x_v1)
        y_bufs = (y_v0, y_v1)
        sem_in = (sem_in0, sem_in1)
        sem_out = (sem_out0, sem_out1)

        def start_in(c):
            n = _SCHED[c]
            sl = pl.ds(base + _OFFS[c], n)
            s = sem_in[c % 2]
            return (pltpu.async_copy(idx_hbm.at[sl],
                                     idx_bufs[c % 2].at[pl.ds(0, n)], s),
                    pltpu.async_copy(x_hbm.at[sl],
                                     x_bufs[c % 2].at[pl.ds(0, n)], s))

        in_flight = [start_in(0)]
        pltpu.sync_copy(shift_hbm, table_v)
        out_flight = [None, None]

        for c in range(_NCH):
            if c + 1 < _NCH:
                in_flight.append(start_in(c + 1))
            for d in in_flight.pop(0):
                d.wait()
            if out_flight[c % 2] is not None:
                out_flight[c % 2].wait()

            ib = idx_bufs[c % 2]
            xb = x_bufs[c % 2]
            yb = y_bufs[c % 2]

            @plsc.parallel_loop(0, _SCHED[c], step=_L, unroll=8)
            def _(i):
                sl = pl.ds(i, _L)
                sv = plsc.load_gather(table_v, [ib[sl]])
                yb[sl] = xb[sl] - sv

            out_flight[c % 2] = pltpu.async_copy(
                yb.at[pl.ds(0, _SCHED[c])],
                out_hbm.at[pl.ds(base + _OFFS[c], _SCHED[c])],
                sem_out[c % 2])

        for d in out_flight:
            if d is not None:
                d.wait()

    return k


_sc_kernel = _build()


def kernel(x, atomic_numbers, shift):
    idx = atomic_numbers.astype(jnp.int32)
    table = jnp.pad(shift.reshape(-1), (0, _TAB - _N_SPECIES))
    return _sc_kernel(x, idx, table)
